# baseline (speedup 1.0000x reference)
import functools

import jax
import jax.numpy as jnp
from jax import lax
from jax.experimental import pallas as pl
from jax.experimental.pallas import tpu as pltpu
from jax.experimental.pallas import tpu_sc as plsc

N = 10000
D = 128
DP = 144
E = 320000
NACC = 10016
C = 96
LANES = 16


def _leaky(x):
    return jnp.maximum(x, 0.2 * x)


def _elu(x):
    return jnp.where(x > 0, x, jnp.exp(jnp.minimum(x, 0.0)) - 1.0)


def _dot(a, b):
    return jnp.dot(a, b, preferred_element_type=jnp.float32)


BR = 2000


def _tc_call(body, out_shapes, *args):
    def spec(shape):
        if shape and shape[0] == N:
            blk = (BR,) + tuple(shape[1:])
            return pl.BlockSpec(blk, lambda i, _nd=len(shape): (i,) + (0,) * (_nd - 1))
        return pl.BlockSpec(tuple(shape), lambda i, _nd=len(shape): (0,) * _nd)

    single = not isinstance(out_shapes, tuple)
    outs = (out_shapes,) if single else out_shapes
    res = pl.pallas_call(
        body,
        grid=(N // BR,),
        in_specs=[spec(a.shape) for a in args],
        out_specs=(tuple(spec(o.shape) for o in outs) if not single
                   else spec(outs[0].shape)),
        out_shape=out_shapes,
    )(*args)
    return res


def _pre_body(xa_ref, xp_ref, W1a_ref, b1a_ref, W1p_ref, b1p_ref,
              Wfc_ref, bfc_ref, Va_ref, Vp_ref,
              xa_out, xp_out, sa_out, sp_out):
    xa = jnp.maximum(_dot(xa_ref[...], W1a_ref[...]) + b1a_ref[...], 0.0)
    xp = jnp.maximum(_dot(xp_ref[...], W1p_ref[...]) + b1p_ref[...], 0.0)
    xa = _dot(xa, Wfc_ref[...]) + bfc_ref[...]
    xp = _dot(xp, Wfc_ref[...]) + bfc_ref[...]
    xa_out[...] = xa
    xp_out[...] = xp
    sa = _dot(xa, Va_ref[...])
    sp = _dot(xp, Vp_ref[...])
    w2a = jnp.exp(_leaky(sa[:, 0:1] + sa[:, 1:2]))
    w2p = jnp.exp(_leaky(sp[:, 0:1] + sp[:, 1:2]))
    sa_out[...] = jnp.concatenate([sa[:, 0:1], sa[:, 2:3], w2a, sa[:, 3:4]], axis=1)
    sp_out[...] = jnp.concatenate([sp[:, 0:1], sp[:, 2:3], w2p, sp[:, 3:4]], axis=1)


def _mid_body(acca_ref, accp_ref, xa_ref, xp_ref, sa_ref, sp_ref,
              Wfc_ref, bfc_ref, Va_ref, Vp_ref,
              xa_out, xp_out, sa_out, sp_out):
    w2a = sa_ref[:, 2:3]
    ha = acca_ref[:, :D] + w2a * xa_ref[...]
    da = acca_ref[:, D:D + 1] + w2a
    xa = _dot(_elu(ha / da), Wfc_ref[...]) + bfc_ref[...]
    w2p = sp_ref[:, 2:3]
    hp = accp_ref[:, :D] + w2p * xp_ref[...]
    dp = accp_ref[:, D:D + 1] + w2p
    xp = _dot(_elu(hp / dp), Wfc_ref[...]) + bfc_ref[...]
    xa_out[...] = xa
    xp_out[...] = xp
    sa = _dot(xa, Va_ref[...])
    sp = _dot(xp, Vp_ref[...])
    w2a1 = jnp.exp(_leaky(sa[:, 0:1] + sa[:, 1:2]))
    sa_out[...] = jnp.concatenate([sa[:, 0:1], sa[:, 1:2], w2a1, sa[:, 3:4]], axis=1)
    sp_out[...] = jnp.concatenate([sp[:, 0:1], sp[:, 0:1], sp[:, 2:3], sp[:, 3:4]], axis=1)


def _post_body(acc0_ref, acc1_ref, xa_ref, sa_ref, W2_ref, b2_ref, out_ref):
    w2 = sa_ref[:, 2:3]
    hh = acc0_ref[:, :D] + acc1_ref[:, :D] + w2 * xa_ref[...]
    dv = acc0_ref[:, D:D + 1] + acc1_ref[:, D:D + 1] + w2
    out_ref[...] = _dot(_elu(hh / dv), W2_ref[...]) + b2_ref[...]


NT1 = N + 16
H1COL = D + 1


@functools.lru_cache(maxsize=None)
def _make_edge_kernel(eps, nt):
    ET = eps // 16
    CT = ET // C
    QT = CT // 4
    RT = NACC // 16
    RB = RT // C
    RREM = RT - RB * C
    mesh = plsc.VectorSubcoreMesh(core_axis_name="c", subcore_axis_name="s",
                                  num_cores=2, num_subcores=16)

    @functools.partial(
        pl.kernel,
        out_type=jax.ShapeDtypeStruct((2, NACC, DP), jnp.float32),
        mesh=mesh,
        compiler_params=pltpu.CompilerParams(
            needs_layout_passes=False, use_tc_tiling_on_sc=False),
        scratch_types=[
            pltpu.VMEM((NT1,), jnp.float32),
            pltpu.VMEM((2, C), jnp.int32),
            pltpu.VMEM((2, C), jnp.int32),
            pltpu.VMEM((2, C), jnp.int32),
            pltpu.VMEM((2, C), jnp.int32),
            pltpu.VMEM((C,), jnp.float32),
            pltpu.VMEM((C, DP), jnp.float32),
            pltpu.VMEM((C, DP), jnp.float32),
            pltpu.VMEM_SHARED((NACC, DP), jnp.float32),
            pltpu.SemaphoreType.DMA,
            pltpu.SemaphoreType.DMA,
            pltpu.SemaphoreType.DMA,
            pltpu.SemaphoreType.DMA,
            pltpu.SemaphoreType.DMA,
            pltpu.SemaphoreType.DMA,
        ],
    )
    def kern(st_hbm, x1_hbm, g_hbm, out_hbm,
             x1_v, stA0, stA1, stB0, stB1, w1_v, rows0_v, rows1_v,
             acc_sh, sem0, sem1, semS0, semS1, semIA, semIB):
        cid = lax.axis_index("c")
        sid = lax.axis_index("s")

        pltpu.sync_copy(x1_hbm.at[pl.ds(cid * NT1, NT1)], x1_v)

        zv = jnp.zeros((LANES,), jnp.float32)

        def zfill(i, _):
            for j in range(DP // LANES):
                rows0_v[i, pl.ds(j * LANES, LANES)] = zv
            return 0

        lax.fori_loop(0, C, zfill, 0)

        def zacc(b, _):
            pltpu.sync_copy(rows0_v, acc_sh.at[pl.ds(sid * RT + b * C, C)])
            return 0

        lax.fori_loop(0, RB, zacc, 0)
        pltpu.sync_copy(rows0_v.at[pl.ds(0, RREM)],
                        acc_sh.at[pl.ds(sid * RT + RB * C, RREM)])
        plsc.subcore_barrier()

        def load_idx(k, st_v):
            pltpu.sync_copy(st_hbm.at[cid, sid * CT + k], st_v)

        def load_idx_async(k, st_v, semI):
            pltpu.async_copy(st_hbm.at[cid, sid * CT + k], st_v, semI)

        def wait_idx(st_v, semI):
            pltpu.make_async_copy(st_hbm.at[cid, 0], st_v, semI).wait()

        def wait_scatter(semS):
            pltpu.make_async_copy(g_hbm.at[pl.ds(0, C)], rows0_v, semS).wait()

        def process(st_v, rows_v, semS):
            @plsc.parallel_loop(0, C // LANES, unroll=2)
            def attn(g):
                off = g * LANES
                lanes = lax.iota(jnp.int32, LANES) + off
                si = st_v[0, pl.ds(off, LANES)]
                ht = plsc.load_gather(
                    rows_v, [lanes, jnp.full((LANES,), H1COL, jnp.int32)])
                z = plsc.load_gather(x1_v, [si]) + ht
                w1_v[pl.ds(off, LANES)] = jnp.exp(_leaky(z))

            @plsc.parallel_loop(0, C, unroll=8)
            def scale(e):
                wb = plsc.load_gather(w1_v, [jnp.full((LANES,), e, jnp.int32)])
                for j in range(DP // LANES):
                    sl = pl.ds(j * LANES, LANES)
                    rows_v[e, sl] = rows_v[e, sl] * wb

            pltpu.async_copy(rows_v, acc_sh.at[st_v.at[0]], semS, add=True)

        load_idx(0, stA0)
        load_idx(1, stA1)
        pltpu.async_copy(g_hbm.at[stA0.at[1]], rows0_v, sem0)

        def quad(q, _):
            c = 4 * q
            @pl.when(q > 0)
            def _():
                wait_scatter(semS1)

            pltpu.async_copy(g_hbm.at[stA1.at[1]], rows1_v, sem1)
            load_idx_async(c + 2, stB0, semIB)
            load_idx_async(c + 3, stB1, semIB)
            pltpu.make_async_copy(g_hbm.at[stA0.at[1]], rows0_v, sem0).wait()
            process(stA0, rows0_v, semS0)
            wait_scatter(semS0)
            wait_idx(stB0, semIB)
            wait_idx(stB1, semIB)
            pltpu.async_copy(g_hbm.at[stB0.at[1]], rows0_v, sem0)
            pltpu.make_async_copy(g_hbm.at[stA1.at[1]], rows1_v, sem1).wait()
            process(stA1, rows1_v, semS1)
            wait_scatter(semS1)

            @pl.when(q + 1 < QT)
            def _():
                load_idx_async(c + 4, stA0, semIA)
                load_idx_async(c + 5, stA1, semIA)

            pltpu.async_copy(g_hbm.at[stB1.at[1]], rows1_v, sem1)
            pltpu.make_async_copy(g_hbm.at[stB0.at[1]], rows0_v, sem0).wait()
            process(stB0, rows0_v, semS0)
            wait_scatter(semS0)

            @pl.when(q + 1 < QT)
            def _():
                wait_idx(stA0, semIA)
                wait_idx(stA1, semIA)
                pltpu.async_copy(g_hbm.at[stA0.at[1]], rows0_v, sem0)

            pltpu.make_async_copy(g_hbm.at[stB1.at[1]], rows1_v, sem1).wait()
            process(stB1, rows1_v, semS1)
            return 0

        lax.fori_loop(0, QT, quad, 0)
        wait_scatter(semS1)
        plsc.subcore_barrier()

        def wb(b, _):
            r0 = sid * RT + b * C
            pltpu.sync_copy(acc_sh.at[pl.ds(r0, C)], rows0_v)
            pltpu.sync_copy(rows0_v, out_hbm.at[cid, pl.ds(r0, C)])
            return 0

        lax.fori_loop(0, RB, wb, 0)
        r0 = sid * RT + RB * C
        pltpu.sync_copy(acc_sh.at[pl.ds(r0, RREM)], rows0_v.at[pl.ds(0, RREM)])
        pltpu.sync_copy(rows0_v.at[pl.ds(0, RREM)], out_hbm.at[cid, pl.ds(r0, RREM)])

    return kern


def _pad_rows(x, h1):
    n = x.shape[0]
    return jnp.concatenate(
        [x, jnp.ones((n, 1), jnp.float32), h1[:, None],
         jnp.zeros((n, DP - D - 2), jnp.float32)], axis=1)


def _x1_tables(x1_c0, x1_c1):
    z = jnp.zeros((NT1 - N,), jnp.float32)
    return jnp.concatenate([x1_c0, z, x1_c1, z])


def _pad_edges(a, eps, fill):
    return jnp.concatenate(
        [a, jnp.full((a.shape[0], eps - a.shape[1]), fill, jnp.int32)], axis=1)


def _st_pack(s, t, eps):
    s = _pad_edges(s, eps, N).reshape(2, eps // C, 1, C)
    t = _pad_edges(t, eps, 0).reshape(2, eps // C, 1, C)
    return jnp.concatenate([s, t], axis=2)


def kernel(x_author, x_paper, edge_index_author_to_paper, edge_index_paper_to_author,
           W1_author, b1_author, W1_paper, b1_paper, Wfc0, bfc0, Wfc1, bfc1,
           a1_h0_author_to_paper, a2_h0_author_to_paper,
           a1_h0_paper_to_author, a2_h0_paper_to_author,
           a1_h1_author_to_paper, a2_h1_author_to_paper,
           a1_h1_paper_to_author, a2_h1_paper_to_author,
           Sp1_h0_author, Sb1_h0_author, Sp2_h0_author,
           Sp1_h0_paper, Sb1_h0_paper, Sp2_h0_paper,
           Sp1_h1_author, Sb1_h1_author, Sp2_h1_author,
           Sp1_h1_paper, Sb1_h1_paper, Sp2_h1_paper,
           W2, b2):
    f32 = jnp.float32
    s_ap = edge_index_author_to_paper[0].astype(jnp.int32)
    t_ap = edge_index_author_to_paper[1].astype(jnp.int32)
    s_pa = edge_index_paper_to_author[0].astype(jnp.int32)
    t_pa = edge_index_paper_to_author[1].astype(jnp.int32)

    zc = jnp.zeros((D,), f32)
    Va0 = jnp.stack([a1_h0_author_to_paper, a2_h0_author_to_paper,
                     a2_h0_paper_to_author, zc], axis=1)
    Vp0 = jnp.stack([a1_h0_paper_to_author, a2_h0_paper_to_author,
                     a2_h0_author_to_paper, zc], axis=1)
    Va1 = jnp.stack([a1_h1_author_to_paper, a2_h1_author_to_paper, zc, zc], axis=1)
    Vp1 = jnp.stack([a2_h1_author_to_paper, zc, zc, zc], axis=1)

    shp = lambda *s: jax.ShapeDtypeStruct(s, f32)

    xa0, xp0, sa0, sp0 = _tc_call(
        _pre_body, (shp(N, D), shp(N, D), shp(N, 4), shp(N, 4)),
        x_author, x_paper, W1_author, b1_author.reshape(1, D),
        W1_paper, b1_paper.reshape(1, D), Wfc0, bfc0.reshape(1, D), Va0, Vp0)

    st0 = _st_pack(jnp.stack([s_ap, s_pa]), jnp.stack([t_ap, t_pa + N]), 325632)
    X10 = _x1_tables(sa0[:, 0], sp0[:, 0])
    G0 = jnp.concatenate(
        [_pad_rows(xp0, sp0[:, 1]), _pad_rows(xa0, sa0[:, 1])], axis=0)
    acc0 = _make_edge_kernel(325632, 2 * N)(st0, X10, G0)

    xa1, xp1, sa1, sp1 = _tc_call(
        _mid_body, (shp(N, D), shp(N, D), shp(N, 4), shp(N, 4)),
        acc0[0, :N], acc0[1, :N], xa0, xp0, sa0, sp0,
        Wfc1, bfc1.reshape(1, D), Va1, Vp1)

    t1_off = t_ap.reshape(2, E // 2) + jnp.array([[0], [N]], jnp.int32)
    st1 = _st_pack(s_ap.reshape(2, E // 2), t1_off, 165888)
    G1 = _pad_rows(xp1, sp1[:, 1])
    G1d = jnp.concatenate([G1, G1], axis=0)
    acc1 = _make_edge_kernel(165888, 2 * N)(
        st1, _x1_tables(sa1[:, 0], sa1[:, 0]), G1d)

    out = _tc_call(
        _post_body, shp(N, D // 2),
        acc1[0, :N], acc1[1, :N], xa1, sa1, W2, b2.reshape(1, D // 2))
    return out


# speedup vs baseline: 1.2698x; 1.2698x over previous
import functools

import jax
import jax.numpy as jnp
from jax import lax
from jax.experimental import pallas as pl
from jax.experimental.pallas import tpu as pltpu
from jax.experimental.pallas import tpu_sc as plsc

N = 10000
D = 128
DP = 144
E = 320000
NACC = 10016
C = 96
LANES = 16


def _leaky(x):
    return jnp.maximum(x, 0.2 * x)


def _elu(x):
    return jnp.where(x > 0, x, jnp.exp(jnp.minimum(x, 0.0)) - 1.0)


def _dot(a, b):
    return jnp.dot(a, b, preferred_element_type=jnp.float32)


BR = 2000


def _tc_call(body, out_shapes, *args):
    def spec(shape):
        if shape and shape[0] == N:
            blk = (BR,) + tuple(shape[1:])
            return pl.BlockSpec(blk, lambda i, _nd=len(shape): (i,) + (0,) * (_nd - 1))
        return pl.BlockSpec(tuple(shape), lambda i, _nd=len(shape): (0,) * _nd)

    single = not isinstance(out_shapes, tuple)
    outs = (out_shapes,) if single else out_shapes
    res = pl.pallas_call(
        body,
        grid=(N // BR,),
        in_specs=[spec(a.shape) for a in args],
        out_specs=(tuple(spec(o.shape) for o in outs) if not single
                   else spec(outs[0].shape)),
        out_shape=out_shapes,
    )(*args)
    return res


def _pre_body(xa_ref, xp_ref, W1a_ref, b1a_ref, W1p_ref, b1p_ref,
              Wfc_ref, bfc_ref, Va_ref, Vp_ref,
              xa_out, xp_out, sa_out, sp_out):
    xa = jnp.maximum(_dot(xa_ref[...], W1a_ref[...]) + b1a_ref[...], 0.0)
    xp = jnp.maximum(_dot(xp_ref[...], W1p_ref[...]) + b1p_ref[...], 0.0)
    xa = _dot(xa, Wfc_ref[...]) + bfc_ref[...]
    xp = _dot(xp, Wfc_ref[...]) + bfc_ref[...]
    xa_out[...] = xa
    xp_out[...] = xp
    sa = _dot(xa, Va_ref[...])
    sp = _dot(xp, Vp_ref[...])
    w2a = jnp.exp(_leaky(sa[:, 0:1] + sa[:, 1:2]))
    w2p = jnp.exp(_leaky(sp[:, 0:1] + sp[:, 1:2]))
    sa_out[...] = jnp.concatenate([sa[:, 0:1], sa[:, 2:3], w2a, sa[:, 3:4]], axis=1)
    sp_out[...] = jnp.concatenate([sp[:, 0:1], sp[:, 2:3], w2p, sp[:, 3:4]], axis=1)


def _mid_body(acca_ref, accp_ref, xa_ref, xp_ref, sa_ref, sp_ref,
              Wfc_ref, bfc_ref, Va_ref, Vp_ref,
              xa_out, xp_out, sa_out, sp_out):
    w2a = sa_ref[:, 2:3]
    ha = acca_ref[:, :D] + w2a * xa_ref[...]
    da = acca_ref[:, D:D + 1] + w2a
    xa = _dot(_elu(ha / da), Wfc_ref[...]) + bfc_ref[...]
    w2p = sp_ref[:, 2:3]
    hp = accp_ref[:, :D] + w2p * xp_ref[...]
    dp = accp_ref[:, D:D + 1] + w2p
    xp = _dot(_elu(hp / dp), Wfc_ref[...]) + bfc_ref[...]
    xa_out[...] = xa
    xp_out[...] = xp
    sa = _dot(xa, Va_ref[...])
    sp = _dot(xp, Vp_ref[...])
    w2a1 = jnp.exp(_leaky(sa[:, 0:1] + sa[:, 1:2]))
    sa_out[...] = jnp.concatenate([sa[:, 0:1], sa[:, 1:2], w2a1, sa[:, 3:4]], axis=1)
    sp_out[...] = jnp.concatenate([sp[:, 0:1], sp[:, 0:1], sp[:, 2:3], sp[:, 3:4]], axis=1)


def _post_body(acc0_ref, acc1_ref, xa_ref, sa_ref, W2_ref, b2_ref, out_ref):
    w2 = sa_ref[:, 2:3]
    hh = acc0_ref[:, :D] + acc1_ref[:, :D] + w2 * xa_ref[...]
    dv = acc0_ref[:, D:D + 1] + acc1_ref[:, D:D + 1] + w2
    out_ref[...] = _dot(_elu(hh / dv), W2_ref[...]) + b2_ref[...]


NT1 = N + 16
H1COL = D + 1


@functools.lru_cache(maxsize=None)
def _make_edge_kernel(eps, nt):
    ET = eps // 16
    CT = ET // C
    CT2 = CT // 2
    RT = NACC // 16
    RB = RT // C
    RREM = RT - RB * C
    mesh = plsc.VectorSubcoreMesh(core_axis_name="c", subcore_axis_name="s",
                                  num_cores=2, num_subcores=16)

    @functools.partial(
        pl.kernel,
        out_type=jax.ShapeDtypeStruct((2, NACC, DP), jnp.float32),
        mesh=mesh,
        compiler_params=pltpu.CompilerParams(
            needs_layout_passes=False, use_tc_tiling_on_sc=False),
        scratch_types=[
            pltpu.VMEM((NT1,), jnp.float32),
            pltpu.VMEM((2, C), jnp.int32),
            pltpu.VMEM((2, C), jnp.int32),
            pltpu.VMEM((C,), jnp.float32),
            pltpu.VMEM((C, DP), jnp.float32),
            pltpu.VMEM((C, DP), jnp.float32),
            pltpu.VMEM_SHARED((NACC, DP), jnp.float32),
            pltpu.SemaphoreType.DMA,
            pltpu.SemaphoreType.DMA,
            pltpu.SemaphoreType.DMA,
            pltpu.SemaphoreType.DMA,
        ],
    )
    def kern(st_hbm, x1_hbm, g_hbm, out_hbm,
             x1_v, st0_v, st1_v, w1_v, rows0_v, rows1_v,
             acc_sh, sem0, sem1, semS0, semS1):
        cid = lax.axis_index("c")
        sid = lax.axis_index("s")

        def load_idx(k, st_v):
            pltpu.sync_copy(st_hbm.at[cid, sid * CT + k], st_v)

        load_idx(0, st0_v)
        pltpu.async_copy(g_hbm.at[st0_v.at[1]], rows0_v, sem0)

        pltpu.sync_copy(x1_hbm.at[pl.ds(cid * NT1, NT1)], x1_v)

        zv = jnp.zeros((LANES,), jnp.float32)

        def zfill(i, _):
            for j in range(DP // LANES):
                rows1_v[i, pl.ds(j * LANES, LANES)] = zv
            return 0

        lax.fori_loop(0, C, zfill, 0)

        def zacc(b, _):
            pltpu.sync_copy(rows1_v, acc_sh.at[pl.ds(sid * RT + b * C, C)])
            return 0

        lax.fori_loop(0, RB, zacc, 0)
        pltpu.sync_copy(rows1_v.at[pl.ds(0, RREM)],
                        acc_sh.at[pl.ds(sid * RT + RB * C, RREM)])
        plsc.subcore_barrier()

        def wait_scatter(semS):
            pltpu.make_async_copy(g_hbm.at[pl.ds(0, C)], rows0_v, semS).wait()

        def process(st_v, rows_v, semS):
            @plsc.parallel_loop(0, C // LANES, unroll=2)
            def attn(g):
                off = g * LANES
                lanes = lax.iota(jnp.int32, LANES) + off
                si = st_v[0, pl.ds(off, LANES)]
                ht = plsc.load_gather(
                    rows_v, [lanes, jnp.full((LANES,), H1COL, jnp.int32)])
                z = plsc.load_gather(x1_v, [si]) + ht
                w1_v[pl.ds(off, LANES)] = jnp.exp(_leaky(z))

            @plsc.parallel_loop(0, C, unroll=8)
            def scale(e):
                wb = plsc.load_gather(w1_v, [jnp.full((LANES,), e, jnp.int32)])
                for j in range(DP // LANES):
                    sl = pl.ds(j * LANES, LANES)
                    rows_v[e, sl] = rows_v[e, sl] * wb

            pltpu.async_copy(rows_v, acc_sh.at[st_v.at[0]], semS, add=True)

        def pair(k2, _):
            @pl.when(k2 > 0)
            def _():
                wait_scatter(semS1)

            load_idx(2 * k2 + 1, st1_v)
            pltpu.async_copy(g_hbm.at[st1_v.at[1]], rows1_v, sem1)
            pltpu.make_async_copy(g_hbm.at[st0_v.at[1]], rows0_v, sem0).wait()
            process(st0_v, rows0_v, semS0)

            @pl.when(k2 + 1 < CT2)
            def _():
                wait_scatter(semS0)
                load_idx(2 * k2 + 2, st0_v)
                pltpu.async_copy(g_hbm.at[st0_v.at[1]], rows0_v, sem0)

            pltpu.make_async_copy(g_hbm.at[st1_v.at[1]], rows1_v, sem1).wait()
            process(st1_v, rows1_v, semS1)
            return 0

        lax.fori_loop(0, CT2, pair, 0)
        wait_scatter(semS0)
        wait_scatter(semS1)
        plsc.subcore_barrier()

        def wb(b, _):
            r0 = sid * RT + b * C
            pltpu.sync_copy(acc_sh.at[pl.ds(r0, C)], rows0_v)
            pltpu.sync_copy(rows0_v, out_hbm.at[cid, pl.ds(r0, C)])
            return 0

        lax.fori_loop(0, RB, wb, 0)
        r0 = sid * RT + RB * C
        pltpu.sync_copy(acc_sh.at[pl.ds(r0, RREM)], rows0_v.at[pl.ds(0, RREM)])
        pltpu.sync_copy(rows0_v.at[pl.ds(0, RREM)], out_hbm.at[cid, pl.ds(r0, RREM)])

    return kern


def _pad_rows(x, h1):
    n = x.shape[0]
    return jnp.concatenate(
        [x, jnp.ones((n, 1), jnp.float32), h1[:, None],
         jnp.zeros((n, DP - D - 2), jnp.float32)], axis=1)


def _x1_tables(x1_c0, x1_c1):
    z = jnp.zeros((NT1 - N,), jnp.float32)
    return jnp.concatenate([x1_c0, z, x1_c1, z])


def _pad_edges(a, eps, fill):
    return jnp.concatenate(
        [a, jnp.full((a.shape[0], eps - a.shape[1]), fill, jnp.int32)], axis=1)


def _st_pack(s, t, eps):
    s = _pad_edges(s, eps, N).reshape(2, eps // C, 1, C)
    t = _pad_edges(t, eps, 0).reshape(2, eps // C, 1, C)
    return jnp.concatenate([s, t], axis=2)


def kernel(x_author, x_paper, edge_index_author_to_paper, edge_index_paper_to_author,
           W1_author, b1_author, W1_paper, b1_paper, Wfc0, bfc0, Wfc1, bfc1,
           a1_h0_author_to_paper, a2_h0_author_to_paper,
           a1_h0_paper_to_author, a2_h0_paper_to_author,
           a1_h1_author_to_paper, a2_h1_author_to_paper,
           a1_h1_paper_to_author, a2_h1_paper_to_author,
           Sp1_h0_author, Sb1_h0_author, Sp2_h0_author,
           Sp1_h0_paper, Sb1_h0_paper, Sp2_h0_paper,
           Sp1_h1_author, Sb1_h1_author, Sp2_h1_author,
           Sp1_h1_paper, Sb1_h1_paper, Sp2_h1_paper,
           W2, b2):
    f32 = jnp.float32
    s_ap = edge_index_author_to_paper[0].astype(jnp.int32)
    t_ap = edge_index_author_to_paper[1].astype(jnp.int32)
    s_pa = edge_index_paper_to_author[0].astype(jnp.int32)
    t_pa = edge_index_paper_to_author[1].astype(jnp.int32)

    zc = jnp.zeros((D,), f32)
    Va0 = jnp.stack([a1_h0_author_to_paper, a2_h0_author_to_paper,
                     a2_h0_paper_to_author, zc], axis=1)
    Vp0 = jnp.stack([a1_h0_paper_to_author, a2_h0_paper_to_author,
                     a2_h0_author_to_paper, zc], axis=1)
    Va1 = jnp.stack([a1_h1_author_to_paper, a2_h1_author_to_paper, zc, zc], axis=1)
    Vp1 = jnp.stack([a2_h1_author_to_paper, zc, zc, zc], axis=1)

    shp = lambda *s: jax.ShapeDtypeStruct(s, f32)

    xa0, xp0, sa0, sp0 = _tc_call(
        _pre_body, (shp(N, D), shp(N, D), shp(N, 4), shp(N, 4)),
        x_author, x_paper, W1_author, b1_author.reshape(1, D),
        W1_paper, b1_paper.reshape(1, D), Wfc0, bfc0.reshape(1, D), Va0, Vp0)

    st0 = _st_pack(jnp.stack([s_ap, s_pa]), jnp.stack([t_ap, t_pa + N]), 322560)
    X10 = _x1_tables(sa0[:, 0], sp0[:, 0])
    G0 = jnp.concatenate(
        [_pad_rows(xp0, sp0[:, 1]), _pad_rows(xa0, sa0[:, 1])], axis=0)
    acc0 = _make_edge_kernel(322560, 2 * N)(st0, X10, G0)

    xa1, xp1, sa1, sp1 = _tc_call(
        _mid_body, (shp(N, D), shp(N, D), shp(N, 4), shp(N, 4)),
        acc0[0, :N], acc0[1, :N], xa0, xp0, sa0, sp0,
        Wfc1, bfc1.reshape(1, D), Va1, Vp1)

    t1_off = t_ap.reshape(2, E // 2) + jnp.array([[0], [N]], jnp.int32)
    st1 = _st_pack(s_ap.reshape(2, E // 2), t1_off, 162816)
    G1 = _pad_rows(xp1, sp1[:, 1])
    G1d = jnp.concatenate([G1, G1], axis=0)
    acc1 = _make_edge_kernel(162816, 2 * N)(
        st1, _x1_tables(sa1[:, 0], sa1[:, 0]), G1d)

    out = _tc_call(
        _post_body, shp(N, D // 2),
        acc1[0, :N], acc1[1, :N], xa1, sa1, W2, b2.reshape(1, D // 2))
    return out


# speedup vs baseline: 1.3481x; 1.0617x over previous
import functools

import jax
import jax.numpy as jnp
from jax import lax
from jax.experimental import pallas as pl
from jax.experimental.pallas import tpu as pltpu
from jax.experimental.pallas import tpu_sc as plsc

N = 10000
D = 128
DP = 144
E = 320000
NACC = 10016
C = 96
LANES = 16


def _leaky(x):
    return jnp.maximum(x, 0.2 * x)


def _elu(x):
    return jnp.where(x > 0, x, jnp.exp(jnp.minimum(x, 0.0)) - 1.0)


def _dot(a, b):
    return jnp.dot(a, b, preferred_element_type=jnp.float32)


BR = 2000


def _tc_call(body, out_shapes, *args):
    def spec(shape):
        if shape and shape[0] == N:
            blk = (BR,) + tuple(shape[1:])
            return pl.BlockSpec(blk, lambda i, _nd=len(shape): (i,) + (0,) * (_nd - 1))
        return pl.BlockSpec(tuple(shape), lambda i, _nd=len(shape): (0,) * _nd)

    single = not isinstance(out_shapes, tuple)
    outs = (out_shapes,) if single else out_shapes
    res = pl.pallas_call(
        body,
        grid=(N // BR,),
        in_specs=[spec(a.shape) for a in args],
        out_specs=(tuple(spec(o.shape) for o in outs) if not single
                   else spec(outs[0].shape)),
        out_shape=out_shapes,
    )(*args)
    return res


def _g_rows(x, h1):
    n = x.shape[0]
    return jnp.concatenate(
        [x, jnp.ones((n, 1), jnp.float32), h1,
         jnp.zeros((n, DP - D - 2), jnp.float32)], axis=1)


def _pre_body(xa_ref, xp_ref, W1a_ref, b1a_ref, W1p_ref, b1p_ref,
              Wfc_ref, bfc_ref, Va_ref, Vp_ref,
              ga_out, gp_out, sa_out, sp_out):
    xa = jnp.maximum(_dot(xa_ref[...], W1a_ref[...]) + b1a_ref[...], 0.0)
    xp = jnp.maximum(_dot(xp_ref[...], W1p_ref[...]) + b1p_ref[...], 0.0)
    xa = _dot(xa, Wfc_ref[...]) + bfc_ref[...]
    xp = _dot(xp, Wfc_ref[...]) + bfc_ref[...]
    sa = _dot(xa, Va_ref[...])
    sp = _dot(xp, Vp_ref[...])
    ga_out[...] = _g_rows(xa, sa[:, 2:3])
    gp_out[...] = _g_rows(xp, sp[:, 2:3])
    w2a = jnp.exp(_leaky(sa[:, 0:1] + sa[:, 1:2]))
    w2p = jnp.exp(_leaky(sp[:, 0:1] + sp[:, 1:2]))
    sa_out[...] = jnp.concatenate([sa[:, 0:1], sa[:, 2:3], w2a, sa[:, 3:4]], axis=1)
    sp_out[...] = jnp.concatenate([sp[:, 0:1], sp[:, 2:3], w2p, sp[:, 3:4]], axis=1)


def _mid_body(acca_ref, accp_ref, ga_ref, gp_ref, sa_ref, sp_ref,
              Wfc_ref, bfc_ref, Va_ref, Vp_ref,
              xa_out, g1_out, sa_out):
    w2a = sa_ref[:, 2:3]
    ha = acca_ref[:, :D] + w2a * ga_ref[:, :D]
    da = acca_ref[:, D:D + 1] + w2a
    xa = _dot(_elu(ha / da), Wfc_ref[...]) + bfc_ref[...]
    w2p = sp_ref[:, 2:3]
    hp = accp_ref[:, :D] + w2p * gp_ref[:, :D]
    dp = accp_ref[:, D:D + 1] + w2p
    xp = _dot(_elu(hp / dp), Wfc_ref[...]) + bfc_ref[...]
    xa_out[...] = xa
    sa = _dot(xa, Va_ref[...])
    sp = _dot(xp, Vp_ref[...])
    g1_out[...] = _g_rows(xp, sp[:, 0:1])
    w2a1 = jnp.exp(_leaky(sa[:, 0:1] + sa[:, 1:2]))
    sa_out[...] = jnp.concatenate([sa[:, 0:1], sa[:, 1:2], w2a1, sa[:, 3:4]], axis=1)


def _post_body(acc0_ref, acc1_ref, xa_ref, sa_ref, W2_ref, b2_ref, out_ref):
    w2 = sa_ref[:, 2:3]
    hh = acc0_ref[:, :D] + acc1_ref[:, :D] + w2 * xa_ref[...]
    dv = acc0_ref[:, D:D + 1] + acc1_ref[:, D:D + 1] + w2
    out_ref[...] = _dot(_elu(hh / dv), W2_ref[...]) + b2_ref[...]


NT1 = N + 16
H1COL = D + 1


@functools.lru_cache(maxsize=None)
def _make_edge_kernel(eps, nt):
    ET = eps // 16
    CT = ET // C
    CT2 = CT // 2
    RT = NACC // 16
    RB = RT // C
    RREM = RT - RB * C
    mesh = plsc.VectorSubcoreMesh(core_axis_name="c", subcore_axis_name="s",
                                  num_cores=2, num_subcores=16)

    @functools.partial(
        pl.kernel,
        out_type=jax.ShapeDtypeStruct((2, NACC, DP), jnp.float32),
        mesh=mesh,
        compiler_params=pltpu.CompilerParams(
            needs_layout_passes=False, use_tc_tiling_on_sc=False),
        scratch_types=[
            pltpu.VMEM((NT1,), jnp.float32),
            pltpu.VMEM((2, C), jnp.int32),
            pltpu.VMEM((2, C), jnp.int32),
            pltpu.VMEM((C,), jnp.float32),
            pltpu.VMEM((C, DP), jnp.float32),
            pltpu.VMEM((C, DP), jnp.float32),
            pltpu.VMEM_SHARED((NACC, DP), jnp.float32),
            pltpu.SemaphoreType.DMA,
            pltpu.SemaphoreType.DMA,
            pltpu.SemaphoreType.DMA,
            pltpu.SemaphoreType.DMA,
        ],
    )
    def kern(st_hbm, x1_hbm, g_hbm, out_hbm,
             x1_v, st0_v, st1_v, w1_v, rows0_v, rows1_v,
             acc_sh, sem0, sem1, semS0, semS1):
        cid = lax.axis_index("c")
        sid = lax.axis_index("s")

        pltpu.sync_copy(x1_hbm.at[pl.ds(cid * NT1, NT1)], x1_v)

        zv = jnp.zeros((LANES,), jnp.float32)

        def zfill(i, _):
            for j in range(DP // LANES):
                rows0_v[i, pl.ds(j * LANES, LANES)] = zv
            return 0

        lax.fori_loop(0, C, zfill, 0)

        def zacc(b, _):
            pltpu.sync_copy(rows0_v, acc_sh.at[pl.ds(sid * RT + b * C, C)])
            return 0

        lax.fori_loop(0, RB, zacc, 0)
        pltpu.sync_copy(rows0_v.at[pl.ds(0, RREM)],
                        acc_sh.at[pl.ds(sid * RT + RB * C, RREM)])
        plsc.subcore_barrier()

        def load_idx(k, st_v):
            pltpu.sync_copy(st_hbm.at[cid, sid * CT + k], st_v)

        def wait_scatter(semS):
            pltpu.make_async_copy(g_hbm.at[pl.ds(0, C)], rows0_v, semS).wait()

        def process(st_v, rows_v, semS):
            @plsc.parallel_loop(0, C // LANES, unroll=2)
            def attn(g):
                off = g * LANES
                lanes = lax.iota(jnp.int32, LANES) + off
                si = st_v[0, pl.ds(off, LANES)]
                ht = plsc.load_gather(
                    rows_v, [lanes, jnp.full((LANES,), H1COL, jnp.int32)])
                z = plsc.load_gather(x1_v, [si]) + ht
                w1_v[pl.ds(off, LANES)] = jnp.exp(_leaky(z))

            @plsc.parallel_loop(0, C, unroll=8)
            def scale(e):
                wb = plsc.load_gather(w1_v, [jnp.full((LANES,), e, jnp.int32)])
                for j in range(DP // LANES):
                    sl = pl.ds(j * LANES, LANES)
                    rows_v[e, sl] = rows_v[e, sl] * wb

            pltpu.async_copy(rows_v, acc_sh.at[st_v.at[0]], semS, add=True)

        load_idx(0, st0_v)
        pltpu.async_copy(g_hbm.at[st0_v.at[1]], rows0_v, sem0)

        def pair(k2, _):
            @pl.when(k2 > 0)
            def _():
                wait_scatter(semS1)

            load_idx(2 * k2 + 1, st1_v)
            pltpu.async_copy(g_hbm.at[st1_v.at[1]], rows1_v, sem1)
            pltpu.make_async_copy(g_hbm.at[st0_v.at[1]], rows0_v, sem0).wait()
            process(st0_v, rows0_v, semS0)

            @pl.when(k2 + 1 < CT2)
            def _():
                wait_scatter(semS0)
                load_idx(2 * k2 + 2, st0_v)
                pltpu.async_copy(g_hbm.at[st0_v.at[1]], rows0_v, sem0)

            pltpu.make_async_copy(g_hbm.at[st1_v.at[1]], rows1_v, sem1).wait()
            process(st1_v, rows1_v, semS1)
            return 0

        lax.fori_loop(0, CT2, pair, 0)
        wait_scatter(semS0)
        wait_scatter(semS1)
        plsc.subcore_barrier()

        def wb(b, _):
            r0 = sid * RT + b * C
            pltpu.sync_copy(acc_sh.at[pl.ds(r0, C)], rows0_v)
            pltpu.sync_copy(rows0_v, out_hbm.at[cid, pl.ds(r0, C)])
            return 0

        lax.fori_loop(0, RB, wb, 0)
        r0 = sid * RT + RB * C
        pltpu.sync_copy(acc_sh.at[pl.ds(r0, RREM)], rows0_v.at[pl.ds(0, RREM)])
        pltpu.sync_copy(rows0_v.at[pl.ds(0, RREM)], out_hbm.at[cid, pl.ds(r0, RREM)])

    return kern


def _pad_rows(x, h1):
    n = x.shape[0]
    return jnp.concatenate(
        [x, jnp.ones((n, 1), jnp.float32), h1[:, None],
         jnp.zeros((n, DP - D - 2), jnp.float32)], axis=1)


def _x1_tables(x1_c0, x1_c1):
    z = jnp.zeros((NT1 - N,), jnp.float32)
    return jnp.concatenate([x1_c0, z, x1_c1, z])


def _pad_edges(a, eps, fill):
    return jnp.concatenate(
        [a, jnp.full((a.shape[0], eps - a.shape[1]), fill, jnp.int32)], axis=1)


def _st_pack(s, t, eps):
    s = _pad_edges(s, eps, N).reshape(2, eps // C, 1, C)
    t = _pad_edges(t, eps, 0).reshape(2, eps // C, 1, C)
    return jnp.concatenate([s, t], axis=2)


def kernel(x_author, x_paper, edge_index_author_to_paper, edge_index_paper_to_author,
           W1_author, b1_author, W1_paper, b1_paper, Wfc0, bfc0, Wfc1, bfc1,
           a1_h0_author_to_paper, a2_h0_author_to_paper,
           a1_h0_paper_to_author, a2_h0_paper_to_author,
           a1_h1_author_to_paper, a2_h1_author_to_paper,
           a1_h1_paper_to_author, a2_h1_paper_to_author,
           Sp1_h0_author, Sb1_h0_author, Sp2_h0_author,
           Sp1_h0_paper, Sb1_h0_paper, Sp2_h0_paper,
           Sp1_h1_author, Sb1_h1_author, Sp2_h1_author,
           Sp1_h1_paper, Sb1_h1_paper, Sp2_h1_paper,
           W2, b2):
    f32 = jnp.float32
    s_ap = edge_index_author_to_paper[0].astype(jnp.int32)
    t_ap = edge_index_author_to_paper[1].astype(jnp.int32)
    s_pa = edge_index_paper_to_author[0].astype(jnp.int32)
    t_pa = edge_index_paper_to_author[1].astype(jnp.int32)

    zc = jnp.zeros((D,), f32)
    Va0 = jnp.stack([a1_h0_author_to_paper, a2_h0_author_to_paper,
                     a2_h0_paper_to_author, zc], axis=1)
    Vp0 = jnp.stack([a1_h0_paper_to_author, a2_h0_paper_to_author,
                     a2_h0_author_to_paper, zc], axis=1)
    Va1 = jnp.stack([a1_h1_author_to_paper, a2_h1_author_to_paper, zc, zc], axis=1)
    Vp1 = jnp.stack([a2_h1_author_to_paper, zc, zc, zc], axis=1)

    shp = lambda *s: jax.ShapeDtypeStruct(s, f32)

    ga0, gp0, sa0, sp0 = _tc_call(
        _pre_body, (shp(N, DP), shp(N, DP), shp(N, 4), shp(N, 4)),
        x_author, x_paper, W1_author, b1_author.reshape(1, D),
        W1_paper, b1_paper.reshape(1, D), Wfc0, bfc0.reshape(1, D), Va0, Vp0)

    st0 = _st_pack(jnp.stack([s_ap, s_pa]), jnp.stack([t_ap, t_pa + N]), 322560)
    X10 = _x1_tables(sa0[:, 0], sp0[:, 0])
    G0 = jnp.concatenate([gp0, ga0], axis=0)
    acc0 = _make_edge_kernel(322560, 2 * N)(st0, X10, G0)

    xa1, g1, sa1 = _tc_call(
        _mid_body, (shp(N, D), shp(N, DP), shp(N, 4)),
        acc0[0, :N], acc0[1, :N], ga0, gp0, sa0, sp0,
        Wfc1, bfc1.reshape(1, D), Va1, Vp1)

    st1 = _st_pack(s_ap.reshape(2, E // 2), t_ap.reshape(2, E // 2), 162816)
    acc1 = _make_edge_kernel(162816, N)(
        st1, _x1_tables(sa1[:, 0], sa1[:, 0]), g1)

    out = _tc_call(
        _post_body, shp(N, D // 2),
        acc1[0, :N], acc1[1, :N], xa1, sa1, W2, b2.reshape(1, D // 2))
    return out


# speedup vs baseline: 1.3768x; 1.0213x over previous
import functools

import jax
import jax.numpy as jnp
from jax import lax
from jax.experimental import pallas as pl
from jax.experimental.pallas import tpu as pltpu
from jax.experimental.pallas import tpu_sc as plsc

N = 10000
D = 128
DP = 144
E = 320000
NACC = 10016
C = 96
LANES = 16


def _leaky(x):
    return jnp.maximum(x, 0.2 * x)


def _elu(x):
    return jnp.where(x > 0, x, jnp.exp(jnp.minimum(x, 0.0)) - 1.0)


def _dot(a, b):
    return jnp.dot(a, b, preferred_element_type=jnp.float32)


BR = 2000


def _tc_call(body, out_shapes, *args):
    def spec(shape):
        if shape and shape[0] == N:
            blk = (BR,) + tuple(shape[1:])
            return pl.BlockSpec(blk, lambda i, _nd=len(shape): (i,) + (0,) * (_nd - 1))
        return pl.BlockSpec(tuple(shape), lambda i, _nd=len(shape): (0,) * _nd)

    arrays = [a[0] if isinstance(a, tuple) else a for a in args]
    specs = [a[1] if isinstance(a, tuple) else spec(a.shape) for a in args]
    single = not isinstance(out_shapes, tuple)
    outs = (out_shapes,) if single else out_shapes
    res = pl.pallas_call(
        body,
        grid=(N // BR,),
        in_specs=specs,
        out_specs=(tuple(spec(o.shape) for o in outs) if not single
                   else spec(outs[0].shape)),
        out_shape=out_shapes,
    )(*arrays)
    return res


def _acc_spec(c):
    return pl.BlockSpec((1, BR, DP), lambda i, _c=c: (_c, i, 0))


def _g_rows(x, h1):
    n = x.shape[0]
    return jnp.concatenate(
        [x, jnp.ones((n, 1), jnp.float32), h1,
         jnp.zeros((n, DP - D - 2), jnp.float32)], axis=1)


def _pre_body(xa_ref, xp_ref, W1a_ref, b1a_ref, W1p_ref, b1p_ref,
              Wfc_ref, bfc_ref, Va_ref, Vp_ref,
              ga_out, gp_out, sa_out, sp_out):
    xa = jnp.maximum(_dot(xa_ref[...], W1a_ref[...]) + b1a_ref[...], 0.0)
    xp = jnp.maximum(_dot(xp_ref[...], W1p_ref[...]) + b1p_ref[...], 0.0)
    xa = _dot(xa, Wfc_ref[...]) + bfc_ref[...]
    xp = _dot(xp, Wfc_ref[...]) + bfc_ref[...]
    sa = _dot(xa, Va_ref[...])
    sp = _dot(xp, Vp_ref[...])
    ga_out[...] = _g_rows(xa, sa[:, 2:3])
    gp_out[...] = _g_rows(xp, sp[:, 2:3])
    w2a = jnp.exp(_leaky(sa[:, 0:1] + sa[:, 1:2]))
    w2p = jnp.exp(_leaky(sp[:, 0:1] + sp[:, 1:2]))
    sa_out[...] = jnp.concatenate([sa[:, 0:1], sa[:, 2:3], w2a, sa[:, 3:4]], axis=1)
    sp_out[...] = jnp.concatenate([sp[:, 0:1], sp[:, 2:3], w2p, sp[:, 3:4]], axis=1)


def _mid_body(acca_ref, accp_ref, ga_ref, gp_ref, sa_ref, sp_ref,
              Wfc_ref, bfc_ref, Va_ref, Vp_ref,
              xa_out, g1_out, sa_out):
    w2a = sa_ref[:, 2:3]
    ha = acca_ref[0, :, :D] + w2a * ga_ref[:, :D]
    da = acca_ref[0, :, D:D + 1] + w2a
    xa = _dot(_elu(ha / da), Wfc_ref[...]) + bfc_ref[...]
    w2p = sp_ref[:, 2:3]
    hp = accp_ref[0, :, :D] + w2p * gp_ref[:, :D]
    dp = accp_ref[0, :, D:D + 1] + w2p
    xp = _dot(_elu(hp / dp), Wfc_ref[...]) + bfc_ref[...]
    xa_out[...] = xa
    sa = _dot(xa, Va_ref[...])
    sp = _dot(xp, Vp_ref[...])
    g1_out[...] = _g_rows(xp, sp[:, 0:1])
    w2a1 = jnp.exp(_leaky(sa[:, 0:1] + sa[:, 1:2]))
    sa_out[...] = jnp.concatenate([sa[:, 0:1], sa[:, 1:2], w2a1, sa[:, 3:4]], axis=1)


def _post_body(acc0_ref, acc1_ref, xa_ref, sa_ref, W2_ref, b2_ref, out_ref):
    w2 = sa_ref[:, 2:3]
    hh = acc0_ref[0, :, :D] + acc1_ref[0, :, :D] + w2 * xa_ref[...]
    dv = acc0_ref[0, :, D:D + 1] + acc1_ref[0, :, D:D + 1] + w2
    out_ref[...] = _dot(_elu(hh / dv), W2_ref[...]) + b2_ref[...]


NT1 = N + 16
H1COL = D + 1


@functools.lru_cache(maxsize=None)
def _make_edge_kernel(eps, nt):
    ET = eps // 16
    CT = ET // C
    CT2 = CT // 2
    RT = NACC // 16
    RB = RT // C
    RREM = RT - RB * C
    mesh = plsc.VectorSubcoreMesh(core_axis_name="c", subcore_axis_name="s",
                                  num_cores=2, num_subcores=16)

    @functools.partial(
        pl.kernel,
        out_type=jax.ShapeDtypeStruct((2, NACC, DP), jnp.float32),
        mesh=mesh,
        compiler_params=pltpu.CompilerParams(
            needs_layout_passes=False, use_tc_tiling_on_sc=False),
        scratch_types=[
            pltpu.VMEM((NT1,), jnp.float32),
            pltpu.VMEM((2, C), jnp.int32),
            pltpu.VMEM((2, C), jnp.int32),
            pltpu.VMEM((C,), jnp.float32),
            pltpu.VMEM((C, DP), jnp.float32),
            pltpu.VMEM((C, DP), jnp.float32),
            pltpu.VMEM_SHARED((NACC, DP), jnp.float32),
            pltpu.SemaphoreType.DMA,
            pltpu.SemaphoreType.DMA,
            pltpu.SemaphoreType.DMA,
            pltpu.SemaphoreType.DMA,
        ],
    )
    def kern(st_hbm, x1_hbm, g_hbm, out_hbm,
             x1_v, st0_v, st1_v, w1_v, rows0_v, rows1_v,
             acc_sh, sem0, sem1, semS0, semS1):
        cid = lax.axis_index("c")
        sid = lax.axis_index("s")

        pltpu.sync_copy(x1_hbm.at[pl.ds(cid * NT1, NT1)], x1_v)

        zv = jnp.zeros((LANES,), jnp.float32)

        def zfill(i, _):
            for j in range(DP // LANES):
                rows0_v[i, pl.ds(j * LANES, LANES)] = zv
            return 0

        lax.fori_loop(0, C, zfill, 0)

        def zacc(b, _):
            pltpu.sync_copy(rows0_v, acc_sh.at[pl.ds(sid * RT + b * C, C)])
            return 0

        lax.fori_loop(0, RB, zacc, 0)
        pltpu.sync_copy(rows0_v.at[pl.ds(0, RREM)],
                        acc_sh.at[pl.ds(sid * RT + RB * C, RREM)])
        plsc.subcore_barrier()

        def load_idx(k, st_v):
            pltpu.sync_copy(st_hbm.at[cid, sid * CT + k], st_v)

        def wait_scatter(semS):
            pltpu.make_async_copy(g_hbm.at[pl.ds(0, C)], rows0_v, semS).wait()

        def process(st_v, rows_v, semS):
            @plsc.parallel_loop(0, C // LANES, unroll=2)
            def attn(g):
                off = g * LANES
                lanes = lax.iota(jnp.int32, LANES) + off
                si = st_v[0, pl.ds(off, LANES)]
                ht = plsc.load_gather(
                    rows_v, [lanes, jnp.full((LANES,), H1COL, jnp.int32)])
                z = plsc.load_gather(x1_v, [si]) + ht
                w1_v[pl.ds(off, LANES)] = jnp.exp(_leaky(z))

            @plsc.parallel_loop(0, C, unroll=8)
            def scale(e):
                wb = plsc.load_gather(w1_v, [jnp.full((LANES,), e, jnp.int32)])
                for j in range(DP // LANES):
                    sl = pl.ds(j * LANES, LANES)
                    rows_v[e, sl] = rows_v[e, sl] * wb

            pltpu.async_copy(rows_v, acc_sh.at[st_v.at[0]], semS, add=True)

        load_idx(0, st0_v)
        pltpu.async_copy(g_hbm.at[st0_v.at[1]], rows0_v, sem0)

        def pair(k2, _):
            @pl.when(k2 > 0)
            def _():
                wait_scatter(semS1)

            load_idx(2 * k2 + 1, st1_v)
            pltpu.async_copy(g_hbm.at[st1_v.at[1]], rows1_v, sem1)
            pltpu.make_async_copy(g_hbm.at[st0_v.at[1]], rows0_v, sem0).wait()
            process(st0_v, rows0_v, semS0)

            @pl.when(k2 + 1 < CT2)
            def _():
                wait_scatter(semS0)
                load_idx(2 * k2 + 2, st0_v)
                pltpu.async_copy(g_hbm.at[st0_v.at[1]], rows0_v, sem0)

            pltpu.make_async_copy(g_hbm.at[st1_v.at[1]], rows1_v, sem1).wait()
            process(st1_v, rows1_v, semS1)
            return 0

        lax.fori_loop(0, CT2, pair, 0)
        wait_scatter(semS0)
        wait_scatter(semS1)
        plsc.subcore_barrier()

        def wb(b, _):
            r0 = sid * RT + b * C
            pltpu.sync_copy(acc_sh.at[pl.ds(r0, C)], rows0_v)
            pltpu.sync_copy(rows0_v, out_hbm.at[cid, pl.ds(r0, C)])
            return 0

        lax.fori_loop(0, RB, wb, 0)
        r0 = sid * RT + RB * C
        pltpu.sync_copy(acc_sh.at[pl.ds(r0, RREM)], rows0_v.at[pl.ds(0, RREM)])
        pltpu.sync_copy(rows0_v.at[pl.ds(0, RREM)], out_hbm.at[cid, pl.ds(r0, RREM)])

    return kern


def _pad_rows(x, h1):
    n = x.shape[0]
    return jnp.concatenate(
        [x, jnp.ones((n, 1), jnp.float32), h1[:, None],
         jnp.zeros((n, DP - D - 2), jnp.float32)], axis=1)


def _x1_tables(x1_c0, x1_c1):
    z = jnp.zeros((NT1 - N,), jnp.float32)
    return jnp.concatenate([x1_c0, z, x1_c1, z])


def _pad_edges(a, eps, fill):
    return jnp.concatenate(
        [a, jnp.full((a.shape[0], eps - a.shape[1]), fill, jnp.int32)], axis=1)


def _st_pack(s, t, eps):
    s = _pad_edges(s, eps, N).reshape(2, eps // C, 1, C)
    t = _pad_edges(t, eps, 0).reshape(2, eps // C, 1, C)
    return jnp.concatenate([s, t], axis=2)


def kernel(x_author, x_paper, edge_index_author_to_paper, edge_index_paper_to_author,
           W1_author, b1_author, W1_paper, b1_paper, Wfc0, bfc0, Wfc1, bfc1,
           a1_h0_author_to_paper, a2_h0_author_to_paper,
           a1_h0_paper_to_author, a2_h0_paper_to_author,
           a1_h1_author_to_paper, a2_h1_author_to_paper,
           a1_h1_paper_to_author, a2_h1_paper_to_author,
           Sp1_h0_author, Sb1_h0_author, Sp2_h0_author,
           Sp1_h0_paper, Sb1_h0_paper, Sp2_h0_paper,
           Sp1_h1_author, Sb1_h1_author, Sp2_h1_author,
           Sp1_h1_paper, Sb1_h1_paper, Sp2_h1_paper,
           W2, b2):
    f32 = jnp.float32
    s_ap = edge_index_author_to_paper[0].astype(jnp.int32)
    t_ap = edge_index_author_to_paper[1].astype(jnp.int32)
    s_pa = edge_index_paper_to_author[0].astype(jnp.int32)
    t_pa = edge_index_paper_to_author[1].astype(jnp.int32)

    zc = jnp.zeros((D,), f32)
    Va0 = jnp.stack([a1_h0_author_to_paper, a2_h0_author_to_paper,
                     a2_h0_paper_to_author, zc], axis=1)
    Vp0 = jnp.stack([a1_h0_paper_to_author, a2_h0_paper_to_author,
                     a2_h0_author_to_paper, zc], axis=1)
    Va1 = jnp.stack([a1_h1_author_to_paper, a2_h1_author_to_paper, zc, zc], axis=1)
    Vp1 = jnp.stack([a2_h1_author_to_paper, zc, zc, zc], axis=1)

    shp = lambda *s: jax.ShapeDtypeStruct(s, f32)

    ga0, gp0, sa0, sp0 = _tc_call(
        _pre_body, (shp(N, DP), shp(N, DP), shp(N, 4), shp(N, 4)),
        x_author, x_paper, W1_author, b1_author.reshape(1, D),
        W1_paper, b1_paper.reshape(1, D), Wfc0, bfc0.reshape(1, D), Va0, Vp0)

    st0 = _st_pack(jnp.stack([s_ap, s_pa]), jnp.stack([t_ap, t_pa + N]), 322560)
    X10 = _x1_tables(sa0[:, 0], sp0[:, 0])
    G0 = jnp.concatenate([gp0, ga0], axis=0)
    acc0 = _make_edge_kernel(322560, 2 * N)(st0, X10, G0)

    xa1, g1, sa1 = _tc_call(
        _mid_body, (shp(N, D), shp(N, DP), shp(N, 4)),
        (acc0, _acc_spec(0)), (acc0, _acc_spec(1)), ga0, gp0, sa0, sp0,
        Wfc1, bfc1.reshape(1, D), Va1, Vp1)

    st1 = _st_pack(s_ap.reshape(2, E // 2), t_ap.reshape(2, E // 2), 162816)
    acc1 = _make_edge_kernel(162816, N)(
        st1, _x1_tables(sa1[:, 0], sa1[:, 0]), g1)

    out = _tc_call(
        _post_body, shp(N, D // 2),
        (acc1, _acc_spec(0)), (acc1, _acc_spec(1)), xa1, sa1,
        W2, b2.reshape(1, D // 2))
    return out
